# baseline (speedup 1.0000x reference)
import functools

import jax
import jax.numpy as jnp
from jax import lax
from jax.experimental import pallas as pl
from jax.experimental.pallas import tpu as pltpu
from jax.experimental.pallas import tpu_sc as plsc

BATCH = 16384
EMB = 64
HID = 10
NC = 2
NS = 16
NW = NC * NS
BPW = BATCH // NW
NG = BPW // 16
CHUNK = 4
NCHUNK = (2 * EMB) // CHUNK

P_W1 = 0
P_B1 = 2 * EMB * HID
P_W2 = P_B1 + HID
P_B2 = P_W2 + HID
P_LEN = 1312

_mesh = plsc.VectorSubcoreMesh(core_axis_name="c", subcore_axis_name="s")


@functools.partial(
    pl.kernel,
    mesh=_mesh,
    out_type=jax.ShapeDtypeStruct((BATCH,), jnp.float32),
    compiler_params=pltpu.CompilerParams(
        needs_layout_passes=False, use_tc_tiling_on_sc=False),
    scratch_types=[
        pltpu.VMEM((BPW // 128, 128), jnp.int32),
        pltpu.VMEM((BPW // 128, 128), jnp.int32),
        pltpu.VMEM((BPW, EMB), jnp.float32),
        pltpu.VMEM((BPW, EMB), jnp.float32),
        pltpu.VMEM((P_LEN,), jnp.float32),
        pltpu.VMEM((HID, BPW), jnp.float32),
        pltpu.VMEM((BPW,), jnp.float32),
        pltpu.SemaphoreType.DMA,
        pltpu.SemaphoreType.DMA,
    ],
)
def _fwd(user_hbm, item_hbm, uemb_hbm, iemb_hbm, params_hbm, out_hbm,
         uidx_v, iidx_v, urows_v, irows_v, params_v, acc_v, out_v,
         sem_u, sem_i):
    wid = lax.axis_index("s") * NC + lax.axis_index("c")
    base = wid * BPW

    pltpu.sync_copy(params_hbm, params_v)
    IDXC = 128
    cps = []
    for j in range(BPW // IDXC):
        s = pl.ds(j * IDXC, IDXC)
        pltpu.sync_copy(user_hbm.at[pl.ds(base + j * IDXC, IDXC)], uidx_v.at[j])
        pltpu.sync_copy(item_hbm.at[pl.ds(base + j * IDXC, IDXC)], iidx_v.at[j])
        cps.append(pltpu.async_copy(
            uemb_hbm.at[uidx_v.at[j]], urows_v.at[s, :], sem_u))
        cps.append(pltpu.async_copy(
            iemb_hbm.at[iidx_v.at[j]], irows_v.at[s, :], sem_i))
    for cp in cps:
        cp.wait()

    def splat(j):
        return plsc.load_gather(params_v, [jnp.full((16,), j, jnp.int32)])

    iota16 = lax.iota(jnp.int32, 16)

    for c in range(NCHUNK):
        k0 = c * CHUNK
        w = [[splat((k0 + kk) * HID + h) for h in range(HID)]
             for kk in range(CHUNK)]
        if c == 0:
            binit = [splat(P_B1 + h) for h in range(HID)]

        def chunk_body(g, carry, k0=k0, w=w, c=c):
            rows = g * 16 + iota16
            xs = []
            for kk in range(CHUNK):
                k = k0 + kk
                if k < EMB:
                    col = jnp.full((16,), k, jnp.int32)
                    xk = plsc.load_gather(urows_v, [rows, col])
                else:
                    col = jnp.full((16,), k - EMB, jnp.int32)
                    xk = plsc.load_gather(irows_v, [rows, col])
                xs.append(jnp.maximum(xk, 0.0))
            for h in range(HID):
                if c == 0:
                    a = binit[h]
                else:
                    a = acc_v[h, pl.ds(g * 16, 16)]
                for kk in range(CHUNK):
                    a = a + xs[kk] * w[kk][h]
                acc_v[h, pl.ds(g * 16, 16)] = a
            return carry

        lax.fori_loop(0, NG, chunk_body, 0)

    w2 = [splat(P_W2 + h) for h in range(HID)]
    b2v = splat(P_B2)

    def out_body(g, carry):
        o = b2v
        for h in range(HID):
            o = o + jnp.maximum(acc_v[h, pl.ds(g * 16, 16)], 0.0) * w2[h]
        out_v[pl.ds(g * 16, 16)] = o
        return carry

    lax.fori_loop(0, NG, out_body, 0)

    pltpu.sync_copy(out_v, out_hbm.at[pl.ds(base, BPW)])


def kernel(user, item, user_emb, item_emb, W1, b1, W2, b2):
    params = jnp.concatenate([
        W1.reshape(-1), b1.reshape(-1), W2.reshape(-1), b2.reshape(-1),
        jnp.zeros((P_LEN - P_B2 - 1,), jnp.float32),
    ])
    out = _fwd(user.astype(jnp.int32), item.astype(jnp.int32),
               user_emb, item_emb, params)
    return out.reshape(BATCH, 1)


# speedup vs baseline: 1.5344x; 1.5344x over previous
import functools

import jax
import jax.numpy as jnp
from jax import lax
from jax.experimental import pallas as pl
from jax.experimental.pallas import tpu as pltpu
from jax.experimental.pallas import tpu_sc as plsc

BATCH = 16384
EMB = 64
HID = 10
NC = 2
NS = 16
NW = NC * NS
BPW = BATCH // NW
CH = 256
NCH = BPW // CH
CG = CH // 16
CHUNK = 4
NKC = EMB // CHUNK

P_B1 = 2 * EMB * HID
P_W2 = P_B1 + HID
P_B2 = P_W2 + HID
P_LEN = 1312

_mesh = plsc.VectorSubcoreMesh(core_axis_name="c", subcore_axis_name="s")


@functools.partial(
    pl.kernel,
    mesh=_mesh,
    out_type=jax.ShapeDtypeStruct((BATCH,), jnp.float32),
    compiler_params=pltpu.CompilerParams(needs_layout_passes=False),
    scratch_types=[
        pltpu.VMEM((BPW,), jnp.int32),
        pltpu.VMEM((BPW,), jnp.int32),
        pltpu.VMEM((CH, EMB), jnp.float32),
        pltpu.VMEM((CH, EMB), jnp.float32),
        pltpu.VMEM((P_LEN,), jnp.float32),
        pltpu.VMEM((HID, BPW), jnp.float32),
        pltpu.VMEM((BPW,), jnp.float32),
        pltpu.SemaphoreType.DMA,
        pltpu.SemaphoreType.DMA,
    ],
)
def _fwd(user_hbm, item_hbm, uemb_hbm, iemb_hbm, params_hbm, out_hbm,
         uidx_v, iidx_v, urows_v, irows_v, params_v, acc_v, out_v,
         sem_u, sem_i):
    wid = lax.axis_index("s") * NC + lax.axis_index("c")
    base = wid * BPW

    pltpu.sync_copy(user_hbm.at[pl.ds(base, BPW)], uidx_v)
    pltpu.sync_copy(item_hbm.at[pl.ds(base, BPW)], iidx_v)
    pltpu.sync_copy(params_hbm, params_v)

    def splat(j):
        return plsc.load_gather(
            params_v, [jnp.full((16,), 1, jnp.int32) * j])

    iota16 = lax.iota(jnp.int32, 16)

    binit = [splat(P_B1 + h) for h in range(HID)]

    def init_body(g, carry):
        for h in range(HID):
            acc_v[h, pl.ds(g * 16, 16)] = binit[h]
        return carry

    lax.fori_loop(0, BPW // 16, init_body, 0)

    def chunk_loop(c, carry):
        def issue_body(g, carry):
            uiv = uidx_v[pl.ds(c * CH + g * 16, 16)]
            iiv = iidx_v[pl.ds(c * CH + g * 16, 16)]
            for j in range(16):
                r = g * 16 + j
                pltpu.async_copy(uemb_hbm.at[uiv[j]], urows_v.at[r], sem_u)
                pltpu.async_copy(iemb_hbm.at[iiv[j]], irows_v.at[r], sem_i)
            return carry

        lax.fori_loop(0, CG, issue_body, 0)

        def drain_body(r, carry):
            pltpu.make_async_copy(uemb_hbm.at[0], urows_v.at[0], sem_u).wait()
            pltpu.make_async_copy(iemb_hbm.at[0], irows_v.at[0], sem_i).wait()
            return carry

        lax.fori_loop(0, CH, drain_body, 0)

        def make_l1(rows_ref, wbase):
            def l1_body(kc, carry):
                k0 = kc * CHUNK
                w = [[splat(wbase + (k0 + kk) * HID + h) for h in range(HID)]
                     for kk in range(CHUNK)]

                def g_body(g, carry):
                    rows = g * 16 + iota16
                    xs = []
                    for kk in range(CHUNK):
                        col = jnp.full((16,), kk, jnp.int32) + k0
                        xk = plsc.load_gather(rows_ref, [rows, col])
                        xs.append(jnp.maximum(xk, 0.0))
                    for h in range(HID):
                        a = acc_v[h, pl.ds(c * CH + g * 16, 16)]
                        for kk in range(CHUNK):
                            a = a + xs[kk] * w[kk][h]
                        acc_v[h, pl.ds(c * CH + g * 16, 16)] = a
                    return carry

                lax.fori_loop(0, CG, g_body, 0)
                return carry

            return l1_body

        lax.fori_loop(0, NKC, make_l1(urows_v, 0), 0)
        lax.fori_loop(0, NKC, make_l1(irows_v, EMB * HID), 0)
        return carry

    lax.fori_loop(0, NCH, chunk_loop, 0)

    w2 = [splat(P_W2 + h) for h in range(HID)]
    b2v = splat(P_B2)

    def out_body(g, carry):
        o = b2v
        for h in range(HID):
            o = o + jnp.maximum(acc_v[h, pl.ds(g * 16, 16)], 0.0) * w2[h]
        out_v[pl.ds(g * 16, 16)] = o
        return carry

    lax.fori_loop(0, BPW // 16, out_body, 0)

    pltpu.sync_copy(out_v, out_hbm.at[pl.ds(base, BPW)])


def kernel(user, item, user_emb, item_emb, W1, b1, W2, b2):
    params = jnp.concatenate([
        W1.reshape(-1), b1.reshape(-1), W2.reshape(-1), b2.reshape(-1),
        jnp.zeros((P_LEN - P_B2 - 1,), jnp.float32),
    ])
    out = _fwd(user.astype(jnp.int32), item.astype(jnp.int32),
               user_emb, item_emb, params)
    return out.reshape(BATCH, 1)
